# Initial kernel scaffold; baseline (speedup 1.0000x reference)
import jax
import jax.numpy as jnp
from jax.experimental import pallas as pl


def kernel(x):
    raise NotImplementedError("write your pallas kernel here")


# speedup vs baseline: 10.2393x; 10.2393x over previous
import functools

import numpy as np
import jax
import jax.numpy as jnp
from jax import lax
from jax.experimental import pallas as pl
from jax.experimental.pallas import tpu as pltpu
from jax.experimental.pallas import tpu_sc as plsc

_B, _C, _H, _W = 8, 96, 224, 224
_N = _H * _W
_R = _B * _C
_K = int(0.3 * _N)
_NV = _N // 16
_M31 = np.int32(0x7FFFFFFF)
_L = 16


def _f2key(v):
    u = plsc.bitcast(v, jnp.int32)
    return u ^ ((u >> 31) & _M31)


def _popcount(m):
    return jnp.max(plsc.all_reduce_population_count(m))


def _find_dstar(hist, nb, kr):
    lanes = lax.iota(jnp.int32, 16)

    def body(j, carry):
        found, dstar, newk, running = carry
        jj = nb // 16 - 1 - j
        h = hist[pl.ds(jj * 16, 16)]
        suf = lax.rev(jnp.cumsum(lax.rev(h, (0,)), axis=0), (0,)) + running
        mask = suf >= kr
        c = _popcount(mask)
        has = jnp.logical_and(found == 0, c > 0)
        sel = lanes == (c - 1)
        s_d = jnp.max(jnp.where(sel, suf, 0))
        h_d = jnp.max(jnp.where(sel, h, 0))
        dstar = jnp.where(has, jj * 16 + c - 1, dstar)
        newk = jnp.where(has, kr - (s_d - h_d), newk)
        found = jnp.where(has, jnp.int32(1), found)
        running = jnp.max(suf)
        return found, dstar, newk, running

    z = jnp.int32(0)
    _, dstar, newk, _ = lax.fori_loop(0, nb // 16, body, (z, z, z, z))
    return dstar, newk


def _zero_hist(hist, nb):
    zeros16 = jnp.zeros((16,), jnp.int32)

    def body(i, _):
        hist[pl.ds(i * 16, 16)] = zeros16
        return 0

    lax.fori_loop(0, nb // 16, body, 0)


def _sc_body(x_hbm, out_hbm, data, cand, hist):
    nc = 2
    rpw = _R // (nc * 16)
    wid = lax.axis_index("s") * nc + lax.axis_index("c")
    lanes = lax.iota(jnp.int32, 16)
    ones16 = jnp.ones((16,), jnp.int32)
    zf16 = jnp.zeros((16,), jnp.float32)

    def row_body(rr, _):
        row = wid * rpw + rr
        pltpu.sync_copy(x_hbm.at[row], data)

        _zero_hist(hist, 4096)

        def h1(i, _):
            key = _f2key(data[pl.ds(i * 16, 16)])
            d = (key >> 20) + 2048
            plsc.addupdate_scatter(hist, [d], ones16)
            return 0

        lax.fori_loop(0, _NV, h1, 0)
        d1, kr1 = _find_dstar(hist, 4096, jnp.int32(_K))

        def c1(i, off):
            key = _f2key(data[pl.ds(i * 16, 16)])
            m = ((key >> 20) + 2048) == d1
            plsc.store_compressed(cand.at[pl.ds(off, 16)], key, mask=m)
            return off + _popcount(m)

        m1 = lax.fori_loop(0, _NV, c1, jnp.int32(0))
        ns1 = (m1 + 15) >> 4

        _zero_hist(hist, 1024)

        def h2(i, _):
            kv = cand[pl.ds(i * 16, 16)]
            valid = (i * 16 + lanes) < m1
            d = (kv >> 10) & jnp.int32(0x3FF)
            plsc.addupdate_scatter(hist, [d], ones16, mask=valid)
            return 0

        lax.fori_loop(0, ns1, h2, 0)
        d2, kr2 = _find_dstar(hist, 1024, kr1)

        def c2(i, off):
            kv = cand[pl.ds(i * 16, 16)]
            valid = (i * 16 + lanes) < m1
            m = jnp.logical_and(((kv >> 10) & jnp.int32(0x3FF)) == d2, valid)
            plsc.store_compressed(cand.at[pl.ds(off, 16)], kv, mask=m)
            return off + _popcount(m)

        m2 = lax.fori_loop(0, ns1, c2, jnp.int32(0))
        ns2 = (m2 + 15) >> 4

        _zero_hist(hist, 1024)

        def h3(i, _):
            kv = cand[pl.ds(i * 16, 16)]
            valid = (i * 16 + lanes) < m2
            d = kv & jnp.int32(0x3FF)
            plsc.addupdate_scatter(hist, [d], ones16, mask=valid)
            return 0

        lax.fori_loop(0, ns2, h3, 0)
        d3, _ = _find_dstar(hist, 1024, kr2)

        thr_key = ((d1 - 2048) << 20) | (d2 << 10) | d3
        tk = jnp.full((16,), thr_key, jnp.int32)
        thr = plsc.bitcast(tk ^ ((tk >> 31) & _M31), jnp.float32)

        def mk(i, _):
            v = data[pl.ds(i * 16, 16)]
            data[pl.ds(i * 16, 16)] = jnp.where(v >= thr, v, zf16)
            return 0

        lax.fori_loop(0, _NV, mk, 0)
        pltpu.sync_copy(data, out_hbm.at[row])
        return 0

    lax.fori_loop(0, rpw, row_body, 0)


def _build():
    mesh = plsc.VectorSubcoreMesh(core_axis_name="c", subcore_axis_name="s")
    return pl.kernel(
        _sc_body,
        out_type=jax.ShapeDtypeStruct((_R, _N), jnp.float32),
        mesh=mesh,
        scratch_types=[
            pltpu.VMEM((_N,), jnp.float32),
            pltpu.VMEM((_N,), jnp.int32),
            pltpu.VMEM((4096,), jnp.int32),
        ],
        compiler_params=pltpu.CompilerParams(needs_layout_passes=False),
    )


def kernel(x):
    out = _build()(x.reshape(_R, _N))
    return out.reshape(_B, _C, _H, _W)


# speedup vs baseline: 14.4201x; 1.4083x over previous
import functools

import numpy as np
import jax
import jax.numpy as jnp
from jax import lax
from jax.experimental import pallas as pl
from jax.experimental.pallas import tpu as pltpu
from jax.experimental.pallas import tpu_sc as plsc

_B, _C, _H, _W = 8, 96, 224, 224
_N = _H * _W
_R = _B * _C
_K = int(0.3 * _N)
_NV = _N // 16
_M31 = np.int32(0x7FFFFFFF)
_L = 16


def _f2key(v):
    u = plsc.bitcast(v, jnp.int32)
    return u ^ ((u >> 31) & _M31)


def _popcount(m):
    return jnp.max(plsc.all_reduce_population_count(m))


def _find_dstar(hist, nb, kr):
    lanes = lax.iota(jnp.int32, 16)

    def body(j, carry):
        found, dstar, newk, running = carry
        jj = nb // 16 - 1 - j
        h = hist[pl.ds(jj * 16, 16)]
        suf = lax.rev(jnp.cumsum(lax.rev(h, (0,)), axis=0), (0,)) + running
        mask = suf >= kr
        c = _popcount(mask)
        has = jnp.logical_and(found == 0, c > 0)
        sel = lanes == (c - 1)
        s_d = jnp.max(jnp.where(sel, suf, 0))
        h_d = jnp.max(jnp.where(sel, h, 0))
        dstar = jnp.where(has, jj * 16 + c - 1, dstar)
        newk = jnp.where(has, kr - (s_d - h_d), newk)
        found = jnp.where(has, jnp.int32(1), found)
        running = jnp.max(suf)
        return found, dstar, newk, running

    z = jnp.int32(0)
    _, dstar, newk, _ = lax.fori_loop(0, nb // 16, body, (z, z, z, z))
    return dstar, newk


def _zero_hist(hist, nb):
    zeros16 = jnp.zeros((16,), jnp.int32)

    @plsc.parallel_loop(0, nb, 16, unroll=4)
    def _(i):
        hist[pl.ds(i, 16)] = zeros16


def _sc_body(x_hbm, out_hbm, data, cand, hist):
    nc = 2
    rpw = _R // (nc * 16)
    wid = lax.axis_index("s") * nc + lax.axis_index("c")
    lanes = lax.iota(jnp.int32, 16)
    ones16 = jnp.ones((16,), jnp.int32)
    zf16 = jnp.zeros((16,), jnp.float32)

    def row_body(rr, _):
        row = wid * rpw + rr
        pltpu.sync_copy(x_hbm.at[row], data)

        _zero_hist(hist, 4096)

        @plsc.parallel_loop(0, _N, 16, unroll=8)
        def _(i):
            key = _f2key(data[pl.ds(i, 16)])
            d = (key >> 20) + 2048
            plsc.addupdate_scatter(hist, [d], ones16)

        d1, kr1 = _find_dstar(hist, 4096, jnp.int32(_K))

        def c1(i, off):
            for u in range(4):
                key = _f2key(data[pl.ds(i * 64 + u * 16, 16)])
                m = ((key >> 20) + 2048) == d1
                plsc.store_compressed(cand.at[pl.ds(off, 16)], key, mask=m)
                off = off + _popcount(m)
            return off

        m1 = lax.fori_loop(0, _NV // 4, c1, jnp.int32(0))
        ns1 = (m1 + 15) >> 4

        _zero_hist(hist, 1024)

        def h2(i, _):
            kv = cand[pl.ds(i * 16, 16)]
            valid = (i * 16 + lanes) < m1
            d = (kv >> 10) & jnp.int32(0x3FF)
            plsc.addupdate_scatter(hist, [d], ones16, mask=valid)
            return 0

        lax.fori_loop(0, ns1, h2, 0)
        d2, kr2 = _find_dstar(hist, 1024, kr1)

        def c2(i, off):
            kv = cand[pl.ds(i * 16, 16)]
            valid = (i * 16 + lanes) < m1
            m = jnp.logical_and(((kv >> 10) & jnp.int32(0x3FF)) == d2, valid)
            plsc.store_compressed(cand.at[pl.ds(off, 16)], kv, mask=m)
            return off + _popcount(m)

        m2 = lax.fori_loop(0, ns1, c2, jnp.int32(0))
        ns2 = (m2 + 15) >> 4

        _zero_hist(hist, 1024)

        def h3(i, _):
            kv = cand[pl.ds(i * 16, 16)]
            valid = (i * 16 + lanes) < m2
            d = kv & jnp.int32(0x3FF)
            plsc.addupdate_scatter(hist, [d], ones16, mask=valid)
            return 0

        lax.fori_loop(0, ns2, h3, 0)
        d3, _ = _find_dstar(hist, 1024, kr2)

        thr_key = ((d1 - 2048) << 20) | (d2 << 10) | d3
        tk = jnp.full((16,), thr_key, jnp.int32)
        thr = plsc.bitcast(tk ^ ((tk >> 31) & _M31), jnp.float32)

        @plsc.parallel_loop(0, _N, 16, unroll=8)
        def _(i):
            v = data[pl.ds(i, 16)]
            data[pl.ds(i, 16)] = jnp.where(v >= thr, v, zf16)
        pltpu.sync_copy(data, out_hbm.at[row])
        return 0

    lax.fori_loop(0, rpw, row_body, 0)


def _build():
    mesh = plsc.VectorSubcoreMesh(core_axis_name="c", subcore_axis_name="s")
    return pl.kernel(
        _sc_body,
        out_type=jax.ShapeDtypeStruct((_R, _N), jnp.float32),
        mesh=mesh,
        scratch_types=[
            pltpu.VMEM((_N,), jnp.float32),
            pltpu.VMEM((_N,), jnp.int32),
            pltpu.VMEM((4096,), jnp.int32),
        ],
        compiler_params=pltpu.CompilerParams(needs_layout_passes=False),
    )


def kernel(x):
    out = _build()(x.reshape(_R, _N))
    return out.reshape(_B, _C, _H, _W)


# speedup vs baseline: 27.9231x; 1.9364x over previous
import functools

import numpy as np
import jax
import jax.numpy as jnp
from jax import lax
from jax.experimental import pallas as pl
from jax.experimental.pallas import tpu as pltpu
from jax.experimental.pallas import tpu_sc as plsc

_B, _C, _H, _W = 8, 96, 224, 224
_N = _H * _W
_R = _B * _C
_K = int(0.3 * _N)
_NV = _N // 16
_M31 = np.int32(0x7FFFFFFF)
_L = 16


def _f2key(v):
    u = plsc.bitcast(v, jnp.int32)
    return u ^ ((u >> 31) & _M31)


def _popcount(m):
    return jnp.max(plsc.all_reduce_population_count(m))


def _walk(histref, nvec, kr):
    lanes = lax.iota(jnp.int32, 16)

    def body(j, carry):
        found, dstar, newk, running = carry
        jj = nvec - 1 - j
        h = histref[pl.ds(jj * 16, 16)]
        suf = lax.rev(jnp.cumsum(lax.rev(h, (0,)), axis=0), (0,)) + running
        mask = suf >= kr
        c = _popcount(mask)
        has = jnp.logical_and(found == 0, c > 0)
        sel = lanes == (c - 1)
        s_d = jnp.max(jnp.where(sel, suf, 0))
        h_d = jnp.max(jnp.where(sel, h, 0))
        dstar = jnp.where(has, jj * 16 + c - 1, dstar)
        newk = jnp.where(has, kr - (s_d - h_d), newk)
        found = jnp.where(has, jnp.int32(1), found)
        running = jnp.max(suf)
        return found, dstar, newk, running

    z = jnp.int32(0)
    _, dstar, newk, _ = lax.fori_loop(0, nvec, body, (z, z, z, z))
    return dstar, newk


def _find_hier(hist, histc, nb, kr):
    lanes = lax.iota(jnp.int32, 16)
    dc, kr2 = _walk(histc, nb // 256, kr)
    h = hist[pl.ds(dc * 16, 16)]
    suf = lax.rev(jnp.cumsum(lax.rev(h, (0,)), axis=0), (0,))
    mask = suf >= kr2
    c = _popcount(mask)
    sel = lanes == (c - 1)
    s_d = jnp.max(jnp.where(sel, suf, 0))
    h_d = jnp.max(jnp.where(sel, h, 0))
    return dc * 16 + c - 1, kr2 - (s_d - h_d)


def _zero_hist(hist, histc, nb):
    zeros16 = jnp.zeros((16,), jnp.int32)

    @plsc.parallel_loop(0, nb, 16, unroll=4)
    def _(i):
        hist[pl.ds(i, 16)] = zeros16

    @plsc.parallel_loop(0, nb // 16, 16, unroll=1)
    def _(i):
        histc[pl.ds(i, 16)] = zeros16


def _sc_body(x_hbm, out_hbm, data, cand, hist, histc):
    nc = 2
    rpw = _R // (nc * 16)
    wid = lax.axis_index("s") * nc + lax.axis_index("c")
    lanes = lax.iota(jnp.int32, 16)
    ones16 = jnp.ones((16,), jnp.int32)
    zf16 = jnp.zeros((16,), jnp.float32)
    zi16 = jnp.zeros((16,), jnp.int32)

    def row_body(rr, _):
        row = wid * rpw + rr
        pltpu.sync_copy(x_hbm.at[row], data)

        _zero_hist(hist, histc, 4096)

        @plsc.parallel_loop(0, _N, 16, unroll=8)
        def _(i):
            key = _f2key(data[pl.ds(i, 16)])
            d = (key >> 20) + 2048
            plsc.addupdate_scatter(hist, [d], ones16)
            plsc.addupdate_scatter(histc, [d >> 4], ones16)

        d1, kr1 = _find_hier(hist, histc, 4096, jnp.int32(_K))

        @plsc.parallel_loop(0, _N, 16, unroll=4, carry=zi16)
        def c1_off(i, off):
            key = _f2key(data[pl.ds(i, 16)])
            m = ((key >> 20) + 2048) == d1
            mi = m.astype(jnp.int32)
            pref = jnp.cumsum(mi, axis=0) - mi
            plsc.store_scatter(cand, [off + pref], key, mask=m)
            return off + plsc.all_reduce_population_count(m)

        m1 = jnp.max(c1_off)
        ns1 = (m1 + 15) >> 4

        _zero_hist(hist, histc, 1024)

        def h2(i, _):
            kv = cand[pl.ds(i * 16, 16)]
            valid = (i * 16 + lanes) < m1
            d = (kv >> 10) & jnp.int32(0x3FF)
            plsc.addupdate_scatter(hist, [d], ones16, mask=valid)
            plsc.addupdate_scatter(histc, [d >> 4], ones16, mask=valid)
            return 0

        lax.fori_loop(0, ns1, h2, 0)
        d2, kr2 = _find_hier(hist, histc, 1024, kr1)

        def c2(i, off):
            kv = cand[pl.ds(i * 16, 16)]
            valid = (i * 16 + lanes) < m1
            m = jnp.logical_and(((kv >> 10) & jnp.int32(0x3FF)) == d2, valid)
            plsc.store_compressed(cand.at[pl.ds(off, 16)], kv, mask=m)
            return off + _popcount(m)

        m2 = lax.fori_loop(0, ns1, c2, jnp.int32(0))
        ns2 = (m2 + 15) >> 4

        _zero_hist(hist, histc, 1024)

        def h3(i, _):
            kv = cand[pl.ds(i * 16, 16)]
            valid = (i * 16 + lanes) < m2
            d = kv & jnp.int32(0x3FF)
            plsc.addupdate_scatter(hist, [d], ones16, mask=valid)
            plsc.addupdate_scatter(histc, [d >> 4], ones16, mask=valid)
            return 0

        lax.fori_loop(0, ns2, h3, 0)
        d3, _ = _find_hier(hist, histc, 1024, kr2)

        thr_key = ((d1 - 2048) << 20) | (d2 << 10) | d3
        tk = jnp.full((16,), thr_key, jnp.int32)
        thr = plsc.bitcast(tk ^ ((tk >> 31) & _M31), jnp.float32)

        @plsc.parallel_loop(0, _N, 16, unroll=8)
        def _(i):
            v = data[pl.ds(i, 16)]
            data[pl.ds(i, 16)] = jnp.where(v >= thr, v, zf16)
        pltpu.sync_copy(data, out_hbm.at[row])
        return 0

    lax.fori_loop(0, rpw, row_body, 0)


def _build():
    mesh = plsc.VectorSubcoreMesh(core_axis_name="c", subcore_axis_name="s")
    return pl.kernel(
        _sc_body,
        out_type=jax.ShapeDtypeStruct((_R, _N), jnp.float32),
        mesh=mesh,
        scratch_types=[
            pltpu.VMEM((_N,), jnp.float32),
            pltpu.VMEM((_N,), jnp.int32),
            pltpu.VMEM((4096,), jnp.int32),
            pltpu.VMEM((256,), jnp.int32),
        ],
        compiler_params=pltpu.CompilerParams(needs_layout_passes=False),
    )


def kernel(x):
    out = _build()(x.reshape(_R, _N))
    return out.reshape(_B, _C, _H, _W)


# speedup vs baseline: 31.3290x; 1.1220x over previous
import functools

import numpy as np
import jax
import jax.numpy as jnp
from jax import lax
from jax.experimental import pallas as pl
from jax.experimental.pallas import tpu as pltpu
from jax.experimental.pallas import tpu_sc as plsc

_B, _C, _H, _W = 8, 96, 224, 224
_N = _H * _W
_R = _B * _C
_K = int(0.3 * _N)
_NV = _N // 16
_M31 = np.int32(0x7FFFFFFF)
_L = 16


def _f2key(v):
    u = plsc.bitcast(v, jnp.int32)
    return u ^ ((u >> 31) & _M31)


def _popcount(m):
    return jnp.max(plsc.all_reduce_population_count(m))


def _walk(histref, nvec, kr):
    lanes = lax.iota(jnp.int32, 16)

    def body(j, carry):
        found, dstar, newk, running = carry
        jj = nvec - 1 - j
        h = histref[pl.ds(jj * 16, 16)]
        suf = lax.rev(jnp.cumsum(lax.rev(h, (0,)), axis=0), (0,)) + running
        mask = suf >= kr
        c = _popcount(mask)
        has = jnp.logical_and(found == 0, c > 0)
        sel = lanes == (c - 1)
        s_d = jnp.max(jnp.where(sel, suf, 0))
        h_d = jnp.max(jnp.where(sel, h, 0))
        dstar = jnp.where(has, jj * 16 + c - 1, dstar)
        newk = jnp.where(has, kr - (s_d - h_d), newk)
        found = jnp.where(has, jnp.int32(1), found)
        running = jnp.max(suf)
        return found, dstar, newk, running

    z = jnp.int32(0)
    _, dstar, newk, _ = lax.fori_loop(0, nvec, body, (z, z, z, z))
    return dstar, newk


def _find_hier(hist, histc, nb, kr):
    lanes = lax.iota(jnp.int32, 16)
    dc, kr2 = _walk(histc, nb // 256, kr)
    h = hist[pl.ds(dc * 16, 16)]
    suf = lax.rev(jnp.cumsum(lax.rev(h, (0,)), axis=0), (0,))
    mask = suf >= kr2
    c = _popcount(mask)
    sel = lanes == (c - 1)
    s_d = jnp.max(jnp.where(sel, suf, 0))
    h_d = jnp.max(jnp.where(sel, h, 0))
    return dc * 16 + c - 1, kr2 - (s_d - h_d)


def _zero_hist(hist, histc, nb):
    zeros16 = jnp.zeros((16,), jnp.int32)

    @plsc.parallel_loop(0, nb, 16, unroll=4)
    def _(i):
        hist[pl.ds(i, 16)] = zeros16

    @plsc.parallel_loop(0, nb // 16, 16, unroll=1)
    def _(i):
        histc[pl.ds(i, 16)] = zeros16


def _build_coarse(hist, histc, nb):
    @plsc.parallel_loop(0, nb // 16, 1, unroll=4)
    def _(i):
        idx = jnp.full((16,), i, jnp.int32)
        plsc.addupdate_scatter(histc, [idx], hist[pl.ds(i * 16, 16)])


def _sc_body(x_hbm, out_hbm, data, cand, hist, histc):
    nc = 2
    rpw = _R // (nc * 16)
    wid = lax.axis_index("s") * nc + lax.axis_index("c")
    lanes = lax.iota(jnp.int32, 16)
    ones16 = jnp.ones((16,), jnp.int32)
    zf16 = jnp.zeros((16,), jnp.float32)
    zi16 = jnp.zeros((16,), jnp.int32)

    def row_body(rr, _):
        row = wid * rpw + rr
        pltpu.sync_copy(x_hbm.at[row], data)

        _zero_hist(hist, histc, 4096)

        @plsc.parallel_loop(0, _N, 16, unroll=8)
        def _(i):
            key = _f2key(data[pl.ds(i, 16)])
            d = (key >> 20) + 2048
            plsc.addupdate_scatter(hist, [d], ones16)

        _build_coarse(hist, histc, 4096)
        d1, kr1 = _find_hier(hist, histc, 4096, jnp.int32(_K))

        @plsc.parallel_loop(0, _N, 16, unroll=4, carry=zi16)
        def c1_off(i, off):
            key = _f2key(data[pl.ds(i, 16)])
            m = ((key >> 20) + 2048) == d1
            mi = m.astype(jnp.int32)
            pref = jnp.cumsum(mi, axis=0) - mi
            plsc.store_scatter(cand, [off + pref], key, mask=m)
            return off + plsc.all_reduce_population_count(m)

        m1 = jnp.max(c1_off)
        ns1 = (m1 + 15) >> 4

        _zero_hist(hist, histc, 1024)

        def h2(i, _):
            kv = cand[pl.ds(i * 16, 16)]
            valid = (i * 16 + lanes) < m1
            d = (kv >> 10) & jnp.int32(0x3FF)
            plsc.addupdate_scatter(hist, [d], ones16, mask=valid)
            return 0

        lax.fori_loop(0, ns1, h2, 0)
        _build_coarse(hist, histc, 1024)
        d2, kr2 = _find_hier(hist, histc, 1024, kr1)

        def c2(i, off):
            kv = cand[pl.ds(i * 16, 16)]
            valid = (i * 16 + lanes) < m1
            m = jnp.logical_and(((kv >> 10) & jnp.int32(0x3FF)) == d2, valid)
            plsc.store_compressed(cand.at[pl.ds(off, 16)], kv, mask=m)
            return off + _popcount(m)

        m2 = lax.fori_loop(0, ns1, c2, jnp.int32(0))
        ns2 = (m2 + 15) >> 4

        _zero_hist(hist, histc, 1024)

        def h3(i, _):
            kv = cand[pl.ds(i * 16, 16)]
            valid = (i * 16 + lanes) < m2
            d = kv & jnp.int32(0x3FF)
            plsc.addupdate_scatter(hist, [d], ones16, mask=valid)
            return 0

        lax.fori_loop(0, ns2, h3, 0)
        _build_coarse(hist, histc, 1024)
        d3, _ = _find_hier(hist, histc, 1024, kr2)

        thr_key = ((d1 - 2048) << 20) | (d2 << 10) | d3
        tk = jnp.full((16,), thr_key, jnp.int32)
        thr = plsc.bitcast(tk ^ ((tk >> 31) & _M31), jnp.float32)

        @plsc.parallel_loop(0, _N, 16, unroll=8)
        def _(i):
            v = data[pl.ds(i, 16)]
            data[pl.ds(i, 16)] = jnp.where(v >= thr, v, zf16)
        pltpu.sync_copy(data, out_hbm.at[row])
        return 0

    lax.fori_loop(0, rpw, row_body, 0)


def _build():
    mesh = plsc.VectorSubcoreMesh(core_axis_name="c", subcore_axis_name="s")
    return pl.kernel(
        _sc_body,
        out_type=jax.ShapeDtypeStruct((_R, _N), jnp.float32),
        mesh=mesh,
        scratch_types=[
            pltpu.VMEM((_N,), jnp.float32),
            pltpu.VMEM((_N,), jnp.int32),
            pltpu.VMEM((4096,), jnp.int32),
            pltpu.VMEM((256,), jnp.int32),
        ],
        compiler_params=pltpu.CompilerParams(needs_layout_passes=False),
    )


def kernel(x):
    out = _build()(x.reshape(_R, _N))
    return out.reshape(_B, _C, _H, _W)
